# baseline (speedup 1.0000x reference)
import functools

import jax
import jax.numpy as jnp
from jax.experimental import pallas as pl

_EPS = 1e-08
_NEG = -3.0e38


def _body(K, mem_ref, keys_ref, beta_ref, wt_ref, bias_ref, out_ref, w_ref):
    mem = mem_ref[0]
    keys = keys_ref[0]
    beta = beta_ref[0]

    k_sq = jnp.sum(keys * keys, axis=-1, keepdims=True)
    k_norm = jnp.maximum(jnp.sqrt(k_sq), _EPS)
    ones_row = jnp.ones((1, mem.shape[1]), dtype=jnp.float32)
    m_sq = jax.lax.dot_general(ones_row, mem * mem,
                               (((1,), (1,)), ((), ())),
                               precision=jax.lax.Precision.HIGHEST,
                               preferred_element_type=jnp.float32)
    m_norm = jnp.maximum(jnp.sqrt(m_sq), _EPS)

    dot = jax.lax.dot_general(keys, mem, (((1,), (1,)), ((), ())),
                              preferred_element_type=jnp.float32)
    denom = k_norm * m_norm + _EPS
    sim = dot / denom * beta.reshape(-1, 1)

    row_max = jnp.max(sim, axis=-1, keepdims=True)
    H_ = sim.shape[0]

    def step(_, carry):
        s, thr, found = carry
        cur = jnp.max(s, axis=-1, keepdims=True)
        hit = s >= cur
        c = jnp.sum(hit.astype(jnp.float32), axis=-1, keepdims=True)
        thr = jnp.where(found < K, cur, thr)
        found = found + c
        s = jnp.where(hit, _NEG, s)
        return s, thr, found

    _, thr, _ = jax.lax.fori_loop(
        0, K, step, (sim, row_max, jnp.zeros((H_, 1), jnp.float32)))

    gt = sim > thr
    eq = sim == thr
    n_gt = jnp.sum(gt.astype(jnp.float32), axis=-1, keepdims=True)
    m = K - n_gt

    N_ = sim.shape[1]
    NB = N_ // 128
    eq3 = eq.astype(jnp.float32).reshape(H_, NB, 128)
    lt_incl = (jax.lax.broadcasted_iota(jnp.int32, (128, 128), 0)
               <= jax.lax.broadcasted_iota(jnp.int32, (128, 128), 1)
               ).astype(jnp.float32)
    intra = jax.lax.dot_general(eq3.reshape(H_ * NB, 128), lt_incl,
                                (((1,), (0,)), ((), ())),
                                preferred_element_type=jnp.float32)
    intra3 = intra.reshape(H_, NB, 128)
    bs = jnp.sum(eq3, axis=-1)
    lt_exc = (jax.lax.broadcasted_iota(jnp.int32, (NB, NB), 0)
              < jax.lax.broadcasted_iota(jnp.int32, (NB, NB), 1)
              ).astype(jnp.float32)
    bpre = jax.lax.dot_general(bs, lt_exc, (((1,), (0,)), ((), ())),
                               preferred_element_type=jnp.float32)
    pre3 = intra3 + bpre[:, :, None]
    sel_eq = jnp.where((eq3 > 0.0) & (pre3 <= m[:, :, None]), 1.0, 0.0)
    selected = gt | (sel_eq.reshape(H_, N_) > 0.0)
    e = jnp.where(selected, jnp.exp(sim - row_max), 0.0)
    denom_s = jnp.sum(e, axis=-1, keepdims=True)
    wts = e / denom_s
    w_ref[0] = wts

    read = jax.lax.dot_general(wts, mem, (((1,), (0,)), ((), ())),
                               preferred_element_type=jnp.float32)
    per_head = jax.lax.dot_general(read, wt_ref[...],
                                   (((1,), (2,)), ((0,), (0,))),
                                   preferred_element_type=jnp.float32)
    out_ref[0] = jnp.sum(per_head, axis=0, keepdims=True) + bias_ref[...]


@jax.jit
def kernel(memory, read_keys, beta, W, b):
    B, N, D = memory.shape
    H = read_keys.shape[1]
    K = 32

    beta3 = beta.reshape(B, 1, H)
    Wt = W.reshape(D, H, D).transpose(1, 0, 2)
    b2 = b.reshape(1, D)

    grid = (B,)
    out_shapes = (
        jax.ShapeDtypeStruct((B, 1, D), jnp.float32),
        jax.ShapeDtypeStruct((B, H, N), jnp.float32),
    )
    read_combined, weights = pl.pallas_call(
        functools.partial(_body, K),
        grid=grid,
        in_specs=[
            pl.BlockSpec((1, N, D), lambda i: (i, 0, 0)),
            pl.BlockSpec((1, H, D), lambda i: (i, 0, 0)),
            pl.BlockSpec((1, 1, H), lambda i: (i, 0, 0)),
            pl.BlockSpec((H, D, D), lambda i: (0, 0, 0)),
            pl.BlockSpec((1, D), lambda i: (0, 0)),
        ],
        out_specs=(
            pl.BlockSpec((1, 1, D), lambda i: (i, 0, 0)),
            pl.BlockSpec((1, H, N), lambda i: (i, 0, 0)),
        ),
        out_shape=out_shapes,
    )(memory, read_keys, beta3, Wt, b2)
    return (read_combined.reshape(B, D), weights)


# speedup vs baseline: 1.3923x; 1.3923x over previous
import functools

import jax
import jax.numpy as jnp
from jax.experimental import pallas as pl
from jax.experimental.pallas import tpu as pltpu

_EPS = 1e-08
_NEG = -3.0e38


def _body(K, mem_ref, keys_ref, beta_ref, wt_ref, bias_ref, out_ref, w_ref,
          osc_ref, cand_ref):
    mem = mem_ref[0]
    keys = keys_ref[0]
    beta = beta_ref[0]
    N, D = mem.shape
    H = keys.shape[0]
    NB = N // 128

    k_sq = jnp.sum(keys * keys, axis=-1, keepdims=True)
    k_norm = jnp.maximum(jnp.sqrt(k_sq), _EPS)

    CH = 8
    rows = N // CH
    msq_parts = []
    for c in range(CH):
        chunk = mem_ref[0, pl.ds(c * rows, rows), :]
        ch3 = chunk.reshape(rows // 128, 128, D)
        msq_parts.append(jnp.sum(ch3 * ch3, axis=-1))
    m_sq = jnp.concatenate(msq_parts, axis=0)
    m_norm = jnp.maximum(jnp.sqrt(m_sq), _EPS)

    dot = jax.lax.dot_general(keys, mem, (((1,), (1,)), ((), ())),
                              preferred_element_type=jnp.float32)
    dot3 = dot.reshape(H, NB, 128)
    denom3 = k_norm[:, :, None] * m_norm[None, :, :] + _EPS
    sim3 = dot3 / denom3 * beta.reshape(H, 1, 1)

    bm0 = jnp.max(sim3, axis=-1)
    row_max = jnp.max(bm0, axis=-1, keepdims=True)

    iob = jax.lax.broadcasted_iota(jnp.int32, (H, NB), 1)

    def blk_step(j, bm):
        cur = jnp.max(bm, axis=-1, keepdims=True)
        candi = jnp.where(bm >= cur, iob, NB)
        csel = jnp.min(candi, axis=-1, keepdims=True)
        onehot = (iob == csel).astype(jnp.float32)
        osc_ref[:, pl.ds(j, 1), :] = onehot[:, None, :]
        bm = jnp.where(onehot > 0.0, _NEG, bm)
        return bm

    jax.lax.fori_loop(0, K, blk_step, bm0)

    for h in range(H):
        cand_ref[h] = jax.lax.dot_general(
            osc_ref[h], sim3[h], (((1,), (0,)), ((), ())),
            precision=jax.lax.Precision.HIGHEST,
            preferred_element_type=jnp.float32)
    cand = cand_ref[...]

    def step(_, carry):
        s, thr, found = carry
        cur = jnp.max(jnp.max(s, axis=-1), axis=-1)[:, None, None]
        hit = s >= cur
        c = jnp.sum(jnp.sum(hit.astype(jnp.float32), axis=-1),
                    axis=-1)[:, None, None]
        thr = jnp.where(found < K, cur, thr)
        found = found + c
        s = jnp.where(hit, _NEG, s)
        return s, thr, found

    _, thr, _ = jax.lax.fori_loop(
        0, K, step, (cand, row_max[:, :, None],
                     jnp.zeros((H, 1, 1), jnp.float32)))

    rm3 = row_max[:, :, None]
    cgt = (cand > thr).astype(jnp.float32)
    ceq = (cand == thr).astype(jnp.float32)
    n_gt = jnp.sum(jnp.sum(cgt, axis=-1), axis=-1)[:, None, None]
    m = K - n_gt
    zsum = jnp.sum(jnp.sum(cgt * jnp.exp(cand - rm3), axis=-1),
                   axis=-1)[:, None, None]
    z = zsum + m * jnp.exp(thr - rm3)

    eq3 = (sim3 == thr).astype(jnp.float32)
    lt_incl = (jax.lax.broadcasted_iota(jnp.int32, (128, 128), 0)
               <= jax.lax.broadcasted_iota(jnp.int32, (128, 128), 1)
               ).astype(jnp.float32)
    intra = jax.lax.dot_general(eq3.reshape(H * NB, 128), lt_incl,
                                (((1,), (0,)), ((), ())),
                                preferred_element_type=jnp.float32)
    intra3 = intra.reshape(H, NB, 128)
    bs = jnp.sum(eq3, axis=-1)
    lt_exc = (jax.lax.broadcasted_iota(jnp.int32, (NB, NB), 0)
              < jax.lax.broadcasted_iota(jnp.int32, (NB, NB), 1)
              ).astype(jnp.float32)
    bpre = jax.lax.dot_general(bs, lt_exc, (((1,), (0,)), ((), ())),
                               preferred_element_type=jnp.float32)
    pre3 = intra3 + bpre[:, :, None]
    selected = (sim3 > thr) | ((eq3 > 0.0) & (pre3 <= m))
    wts3 = jnp.where(selected, jnp.exp(sim3 - rm3), 0.0) / z
    w_ref[0] = wts3

    wts = wts3.reshape(H, N)
    read = jax.lax.dot_general(wts, mem, (((1,), (0,)), ((), ())),
                               preferred_element_type=jnp.float32)
    per_head = jax.lax.dot_general(read, wt_ref[...],
                                   (((1,), (2,)), ((0,), (0,))),
                                   preferred_element_type=jnp.float32)
    out_ref[0] = jnp.sum(per_head, axis=0, keepdims=True) + bias_ref[...]


@jax.jit
def kernel(memory, read_keys, beta, W, b):
    B, N, D = memory.shape
    H = read_keys.shape[1]
    K = 32
    NB = N // 128

    beta3 = beta.reshape(B, 1, H)
    Wt = W.reshape(D, H, D).transpose(1, 0, 2)
    b2 = b.reshape(1, D)

    grid = (B,)
    out_shapes = (
        jax.ShapeDtypeStruct((B, 1, D), jnp.float32),
        jax.ShapeDtypeStruct((B, H, NB, 128), jnp.float32),
    )
    read_combined, weights = pl.pallas_call(
        functools.partial(_body, K),
        grid=grid,
        in_specs=[
            pl.BlockSpec((1, N, D), lambda i: (i, 0, 0)),
            pl.BlockSpec((1, H, D), lambda i: (i, 0, 0)),
            pl.BlockSpec((1, 1, H), lambda i: (i, 0, 0)),
            pl.BlockSpec((H, D, D), lambda i: (0, 0, 0)),
            pl.BlockSpec((1, D), lambda i: (0, 0)),
        ],
        out_specs=(
            pl.BlockSpec((1, 1, D), lambda i: (i, 0, 0)),
            pl.BlockSpec((1, H, NB, 128), lambda i: (i, 0, 0, 0)),
        ),
        out_shape=out_shapes,
        scratch_shapes=[
            pltpu.VMEM((H, K, NB), jnp.float32),
            pltpu.VMEM((H, K, 128), jnp.float32),
        ],
        compiler_params=pltpu.CompilerParams(
            vmem_limit_bytes=60 * 1024 * 1024),
    )(memory, read_keys, beta3, Wt, b2)
    return (read_combined.reshape(B, D), weights.reshape(B, H, N))
